# Initial kernel scaffold; baseline (speedup 1.0000x reference)
import jax
import jax.numpy as jnp
from jax.experimental import pallas as pl


def kernel(x, gate_w, wv1, w2):
    raise NotImplementedError("write your pallas kernel here")


# speedup vs baseline: 1.5791x; 1.5791x over previous
import functools

import jax
import jax.numpy as jnp
from jax import lax
from jax.experimental import pallas as pl
from jax.experimental.pallas import tpu as pltpu
from jax.experimental.pallas import tpu_sc as plsc

DM = 2048
FFN = 3584
NE = 8
NT = 2048
BR = 256
NB = (NT * 2) // BR + NE
GP = NB * BR
FB = 896
NF = FFN // FB

NW = 32
TPW = NT // NW


def _router_body(x_ref, gw_ref, p0_ref, p1_ref, w0_ref, w1_ref, be_ref):
    x = x_ref[...]
    gw = gw_ref[...]
    logits = lax.dot_general(x, gw, (((1,), (1,)), ((), ())),
                             preferred_element_type=jnp.float32)
    m = jnp.max(logits, axis=-1, keepdims=True)
    e = jnp.exp(logits - m)
    probs = e / jnp.sum(e, axis=-1, keepdims=True)

    lane = lax.broadcasted_iota(jnp.int32, (NT, NE), 1)
    m0 = jnp.max(probs, axis=-1, keepdims=True)
    i0 = jnp.min(jnp.where(probs == m0, lane, NE), axis=-1, keepdims=True)
    probs2 = jnp.where(lane == i0, -1.0, probs)
    m1 = jnp.max(probs2, axis=-1, keepdims=True)
    i1 = jnp.min(jnp.where(probs2 == m1, lane, NE), axis=-1, keepdims=True)
    denom = jnp.abs(m0) + jnp.abs(m1)
    w0 = m0 / denom
    w1 = m1 / denom

    oh0 = (lane == i0).astype(jnp.float32)
    oh1 = (lane == i1).astype(jnp.float32)

    CH = 256
    r = lax.broadcasted_iota(jnp.int32, (CH, CH), 0)
    c = lax.broadcasted_iota(jnp.int32, (CH, CH), 1)
    lt = (r > c).astype(jnp.float32)

    def strict_ranks(oh, run):
        outs = []
        for ci in range(NT // CH):
            blk = oh[ci * CH:(ci + 1) * CH, :]
            within = lax.dot_general(lt, blk, (((1,), (0,)), ((), ())),
                                     preferred_element_type=jnp.float32)
            outs.append(within + run)
            run = run + jnp.sum(blk, axis=0, keepdims=True)
        return jnp.concatenate(outs, axis=0), run

    run = jnp.zeros((1, NE), jnp.float32)
    r0, run = strict_ranks(oh0, run)
    r1, run = strict_ranks(oh1, run)
    counts = run

    nblk = jnp.ceil(counts / BR)
    r8 = lax.broadcasted_iota(jnp.int32, (NE, NE), 0)
    c8 = lax.broadcasted_iota(jnp.int32, (NE, NE), 1)
    ut = (r8 < c8).astype(jnp.float32)
    bs = lax.dot_general(nblk, ut, (((1,), (0,)), ((), ())),
                         preferred_element_type=jnp.float32)
    poff = bs * BR

    rank0 = jnp.sum(r0 * oh0, axis=-1, keepdims=True)
    off0 = jnp.sum(poff * oh0, axis=-1, keepdims=True)
    rank1 = jnp.sum(r1 * oh1, axis=-1, keepdims=True)
    off1 = jnp.sum(poff * oh1, axis=-1, keepdims=True)
    p0_ref[...] = (rank0 + off0).astype(jnp.int32)
    p1_ref[...] = (rank1 + off1).astype(jnp.int32)
    w0_ref[...] = jnp.broadcast_to(w0, (NT, 16))
    w1_ref[...] = jnp.broadcast_to(w1, (NT, 16))

    bidx = lax.broadcasted_iota(jnp.int32, (NB, NE), 0).astype(jnp.float32)
    bse = jnp.broadcast_to(bs, (NB, NE))
    be_ref[...] = (jnp.sum((bse <= bidx).astype(jnp.int32), axis=-1,
                           keepdims=True) - 1)


_router = pl.pallas_call(
    _router_body,
    out_shape=(
        jax.ShapeDtypeStruct((NT, 1), jnp.int32),
        jax.ShapeDtypeStruct((NT, 1), jnp.int32),
        jax.ShapeDtypeStruct((NT, 16), jnp.float32),
        jax.ShapeDtypeStruct((NT, 16), jnp.float32),
        jax.ShapeDtypeStruct((NB, 1), jnp.int32),
    ),
)


@functools.lru_cache(maxsize=None)
def _make_scatter():
    mesh = plsc.VectorSubcoreMesh(core_axis_name="c", subcore_axis_name="s", num_cores=2, num_subcores=16)

    @functools.partial(
        pl.kernel, mesh=mesh,
        out_type=jax.ShapeDtypeStruct((GP, DM), jnp.float32),
        scratch_types=[
            pltpu.VMEM((16,), jnp.int32),
            pltpu.VMEM((16, DM), jnp.float32),
            pltpu.SemaphoreType.DMA,
        ],
    )
    def scat(x_hbm, p0_hbm, p1_hbm, xs_hbm, idx_v, rows_v, sem):
        wid = lax.axis_index("s") * 2 + lax.axis_index("c")
        base = wid * TPW
        for c in range(TPW // 16):
            pltpu.sync_copy(x_hbm.at[pl.ds(base + c * 16, 16)], rows_v)
            pltpu.sync_copy(p0_hbm.at[wid, c], idx_v)
            pltpu.async_copy(rows_v, xs_hbm.at[idx_v], sem).wait()
            pltpu.sync_copy(p1_hbm.at[wid, c], idx_v)
            pltpu.async_copy(rows_v, xs_hbm.at[idx_v], sem).wait()

    return scat


def _m1_body(be_ref, xs_ref, w1_ref, v1_ref, g_ref):
    xsb = xs_ref[...]
    w1 = w1_ref[0]
    v1 = v1_ref[0]
    h = lax.dot_general(xsb, w1, (((1,), (1,)), ((), ())),
                        preferred_element_type=jnp.float32)
    v = lax.dot_general(xsb, v1, (((1,), (1,)), ((), ())),
                        preferred_element_type=jnp.float32)
    g_ref[...] = h * lax.logistic(h) * v


_m1 = pl.pallas_call(
    _m1_body,
    grid_spec=pltpu.PrefetchScalarGridSpec(
        num_scalar_prefetch=1,
        grid=(NF, NB),
        in_specs=[
            pl.BlockSpec((BR, DM), lambda f, i, be: (i, 0)),
            pl.BlockSpec((1, FB, DM), lambda f, i, be: (be[i], f, 0)),
            pl.BlockSpec((1, FB, DM), lambda f, i, be: (be[i], NF + f, 0)),
        ],
        out_specs=pl.BlockSpec((BR, FB), lambda f, i, be: (i, f)),
    ),
    out_shape=jax.ShapeDtypeStruct((GP, FFN), jnp.float32),
)


def _m2_body(be_ref, g_ref, w2_ref, y_ref):
    gb = g_ref[...]
    w2 = w2_ref[0]
    y_ref[...] = lax.dot_general(gb, w2, (((1,), (1,)), ((), ())),
                                 preferred_element_type=jnp.float32)


DB = 1024
ND = DM // DB

_m2 = pl.pallas_call(
    _m2_body,
    grid_spec=pltpu.PrefetchScalarGridSpec(
        num_scalar_prefetch=1,
        grid=(ND, NB),
        in_specs=[
            pl.BlockSpec((BR, FFN), lambda d, i, be: (i, 0)),
            pl.BlockSpec((1, DB, FFN), lambda d, i, be: (be[i], d, 0)),
        ],
        out_specs=pl.BlockSpec((BR, DB), lambda d, i, be: (i, d)),
    ),
    out_shape=jax.ShapeDtypeStruct((GP, DM), jnp.float32),
)


@functools.lru_cache(maxsize=None)
def _make_combine():
    mesh = plsc.VectorSubcoreMesh(core_axis_name="c", subcore_axis_name="s", num_cores=2, num_subcores=16)
    GT = 8

    @functools.partial(
        pl.kernel, mesh=mesh,
        out_type=jax.ShapeDtypeStruct((NT, DM), jnp.float32),
        scratch_types=[
            pltpu.VMEM((GT,), jnp.int32),
            pltpu.VMEM((GT,), jnp.int32),
            pltpu.VMEM((GT, DM), jnp.float32),
            pltpu.VMEM((GT, DM), jnp.float32),
            pltpu.VMEM((GT, DM), jnp.float32),
            pltpu.VMEM((GT, 16), jnp.float32),
            pltpu.VMEM((GT, 16), jnp.float32),
            pltpu.SemaphoreType.DMA,
            pltpu.SemaphoreType.DMA,
        ],
    )
    def comb(ys_hbm, p0_hbm, p1_hbm, wb0_hbm, wb1_hbm, out_hbm,
             i0_v, i1_v, r0_v, r1_v, o_v, w0_v, w1_v, s0, s1):
        wid = lax.axis_index("s") * 2 + lax.axis_index("c")
        base = wid * TPW
        for cg in range(TPW // GT):
            pltpu.sync_copy(p0_hbm.at[wid, cg], i0_v)
            pltpu.sync_copy(p1_hbm.at[wid, cg], i1_v)
            c0 = pltpu.async_copy(ys_hbm.at[i0_v], r0_v, s0)
            c1 = pltpu.async_copy(ys_hbm.at[i1_v], r1_v, s1)
            pltpu.sync_copy(wb0_hbm.at[wid, cg], w0_v)
            pltpu.sync_copy(wb1_hbm.at[wid, cg], w1_v)
            c0.wait()
            c1.wait()
            for mi in range(GT):
                wv0 = w0_v[mi]
                wv1 = w1_v[mi]

                def body(ci, _):
                    a = r0_v[mi, pl.ds(ci * 16, 16)]
                    b = r1_v[mi, pl.ds(ci * 16, 16)]
                    o_v[mi, pl.ds(ci * 16, 16)] = wv0 * a + wv1 * b
                    return 0

                lax.fori_loop(0, DM // 16, body, 0)
            pltpu.sync_copy(o_v, out_hbm.at[pl.ds(base + cg * GT, GT)])

    return comb


def kernel(x, gate_w, wv1, w2):
    pos0, pos1, w0b, w1b, be = _router(x, gate_w)
    be_flat = be.reshape(NB)
    xs = _make_scatter()(x,
                  pos0.reshape(NW, TPW // 16, 16),
                  pos1.reshape(NW, TPW // 16, 16))
    g = _m1(be_flat, xs, wv1, wv1)
    ys = _m2(be_flat, g, w2)
    out = _make_combine()(ys,
                   pos0.reshape(NW, TPW // 8, 8),
                   pos1.reshape(NW, TPW // 8, 8),
                   w0b.reshape(NW, TPW // 8, 8, 16),
                   w1b.reshape(NW, TPW // 8, 8, 16))
    return out
